# baseline (speedup 1.0000x reference)
import functools
import math

import jax
import jax.numpy as jnp
import numpy as np
from jax import lax
from jax.experimental import pallas as pl
from jax.experimental.pallas import tpu as pltpu
from jax.experimental.pallas import tpu_sc as plsc

INPUT_DIM = 3
NUM_LEVELS = 16
LEVEL_DIM = 2
BASE_RESOLUTION = 16
LOG2_HASHMAP_SIZE = 19

NC = 2
NS = 16
NW = NC * NS
LANES = 16


def _level_tables():
    offsets = []
    offset = 0
    max_params = 2 ** LOG2_HASHMAP_SIZE
    for i in range(NUM_LEVELS):
        resolution = int(np.ceil(BASE_RESOLUTION * 2.0 ** i))
        params_in_level = min(max_params, resolution ** INPUT_DIM)
        params_in_level = int(np.ceil(params_in_level / 8) * 8)
        offsets.append(offset)
        offset += params_in_level
    offsets.append(offset)

    params = []
    for lvl in range(NUM_LEVELS):
        size = offsets[lvl + 1] - offsets[lvl]
        scale = 2.0 ** lvl * BASE_RESOLUTION - 1.0
        resolution = int(math.ceil(scale)) + 1
        stride = 1
        coeffs = []
        use_stride = []
        for _ in range(INPUT_DIM):
            use_stride.append(stride <= size)
            coeffs.append(stride % (2 ** 32))
            stride = (stride * (resolution + 1)) % (2 ** 32)
        hashed = stride > size
        if hashed:
            c1 = int(np.int32(np.uint32(2654435761)))
            c2 = int(np.int32(np.uint32(805459861)))
            mode_add = False
        else:
            assert all(use_stride)
            c1 = int(np.int32(np.uint32(coeffs[1])))
            c2 = int(np.int32(np.uint32(coeffs[2])))
            mode_add = True
        params.append(dict(scale=float(scale), mask=size - 1,
                           off=offsets[lvl], add=mode_add, c1=c1, c2=c2))
    return params


_LEVELS = _level_tables()


def _make_grid_kernel(batch):
    chunk = batch // NW
    n = 512
    rw = 8
    assert chunk % n == 0
    nsub = chunk // n
    groups = n // LANES
    m = 8 * n // 128

    f32 = jnp.float32
    i32 = jnp.int32

    def body(xh, yh, zh, eh, oh,
             xb, yb, zb, fb, idx0, idx1, rows0, rows1, acc0, acc1,
             gsem0, gsem1, osem0, osem1):
        cid = lax.axis_index("c")
        sid = lax.axis_index("s")
        wid = sid * NC + cid
        base_w = wid * chunk
        iota = lax.iota(i32, LANES)
        ch0 = jnp.full((LANES,), 0, dtype=i32)
        ch1 = jnp.full((LANES,), 1, dtype=i32)
        idxb = (idx0, idx1)
        rowsb = (rows0, rows1)
        accb = (acc0, acc1)
        gsem = (gsem0, gsem1)
        osem = (osem0, osem1)

        def subchunk(s, carry):
            base = base_w + s * n
            pltpu.sync_copy(xh.at[pl.ds(base, n)], xb)
            pltpu.sync_copy(yh.at[pl.ds(base, n)], yb)
            pltpu.sync_copy(zh.at[pl.ds(base, n)], zb)

            def tbody(g, c):
                o = g * LANES
                for ref in (xb, yb, zb):
                    v = ref[pl.ds(o, LANES)]
                    ref[pl.ds(o, LANES)] = (v + f32(1.0)) * f32(0.5)
                return c
            lax.fori_loop(0, groups, tbody, 0)

            def compute_idx(lvl, p):
                prm = _LEVELS[lvl]
                scale = f32(prm["scale"])
                half = f32(0.5)
                c1 = i32(prm["c1"])
                c2 = i32(prm["c2"])
                mask = i32(prm["mask"])
                off = i32(prm["off"])
                iref = idxb[p]

                def cbody(g, c):
                    o = g * LANES
                    px = xb[pl.ds(o, LANES)] * scale + half
                    py = yb[pl.ds(o, LANES)] * scale + half
                    pz = zb[pl.ds(o, LANES)] * scale + half
                    pix = px.astype(i32)
                    piy = py.astype(i32)
                    piz = pz.astype(i32)
                    fb[p, 0, pl.ds(o, LANES)] = px - pix.astype(f32)
                    fb[p, 1, pl.ds(o, LANES)] = py - piy.astype(f32)
                    fb[p, 2, pl.ds(o, LANES)] = pz - piz.astype(f32)
                    ax = (pix, pix + i32(1))
                    by = (piy * c1, piy * c1 + c1)
                    cz = (piz * c2, piz * c2 + c2)
                    for c8 in range(8):
                        a = ax[c8 & 1]
                        b = by[(c8 >> 1) & 1]
                        cc = cz[(c8 >> 2) & 1]
                        if prm["add"]:
                            h = a + b + cc
                        else:
                            h = a ^ b ^ cc
                        iref[pl.ds(c8 * n + o, LANES)] = (h & mask) + off
                    return c
                lax.fori_loop(0, groups, cbody, 0)

            def accumulate(lvl, p):
                rref = rowsb[p]
                aref = accb[p]

                def abody(g, c):
                    o = g * LANES
                    fx = fb[p, 0, pl.ds(o, LANES)]
                    fy = fb[p, 1, pl.ds(o, LANES)]
                    fz = fb[p, 2, pl.ds(o, LANES)]
                    gx = f32(1.0) - fx
                    gy = f32(1.0) - fy
                    gz = f32(1.0) - fz
                    wxy = (gx * gy, fx * gy, gx * fy, fx * fy)
                    jv = o + iota
                    a0 = None
                    a1 = None
                    for c8 in range(8):
                        w = wxy[c8 & 3] * (gz if c8 < 4 else fz)
                        v0 = plsc.load_gather(rref, [jv + c8 * n, ch0])
                        v1 = plsc.load_gather(rref, [jv + c8 * n, ch1])
                        t0 = w * v0
                        t1 = w * v1
                        a0 = t0 if a0 is None else a0 + t0
                        a1 = t1 if a1 is None else a1 + t1
                    aref[0, pl.ds(o, LANES)] = a0
                    aref[1, pl.ds(o, LANES)] = a1
                    return c
                lax.fori_loop(0, groups, abody, 0)

            ghandles = [None, None]
            ohandles = [None, None]
            for lvl in range(NUM_LEVELS):
                p = lvl & 1
                compute_idx(lvl, p)
                ghandles[p] = pltpu.async_copy(eh.at[idxb[p]], rowsb[p],
                                               gsem[p])
                if lvl > 0:
                    q = (lvl - 1) & 1
                    ghandles[q].wait()
                    if lvl >= 3:
                        ohandles[q].wait()
                    accumulate(lvl - 1, q)
                    ohandles[q] = pltpu.async_copy(
                        accb[q],
                        oh.at[pl.ds(2 * (lvl - 1), 2), pl.ds(base, n)],
                        osem[q])
            ghandles[1].wait()
            ohandles[1].wait()
            accumulate(NUM_LEVELS - 1, 1)
            ohandles[1] = pltpu.async_copy(
                accb[1],
                oh.at[pl.ds(2 * (NUM_LEVELS - 1), 2), pl.ds(base, n)],
                osem[1])
            ohandles[0].wait()
            ohandles[1].wait()
            return carry

        lax.fori_loop(0, nsub, subchunk, 0)

    mesh = plsc.VectorSubcoreMesh(core_axis_name="c", subcore_axis_name="s")
    return pl.kernel(
        body,
        out_type=jax.ShapeDtypeStruct((NUM_LEVELS * LEVEL_DIM, batch), f32),
        mesh=mesh,
        compiler_params=pltpu.CompilerParams(
            needs_layout_passes=False,
            use_tc_tiling_on_sc=False,
        ),
        scratch_types=[
            pltpu.VMEM((n,), f32),
            pltpu.VMEM((n,), f32),
            pltpu.VMEM((n,), f32),
            pltpu.VMEM((2, 3, n), f32),
            pltpu.VMEM((8 * n,), i32),
            pltpu.VMEM((8 * n,), i32),
            pltpu.VMEM((8 * n, rw), f32),
            pltpu.VMEM((8 * n, rw), f32),
            pltpu.VMEM((LEVEL_DIM, n), f32),
            pltpu.VMEM((LEVEL_DIM, n), f32),
            pltpu.SemaphoreType.DMA,
            pltpu.SemaphoreType.DMA,
            pltpu.SemaphoreType.DMA,
            pltpu.SemaphoreType.DMA,
        ],
    )


@jax.jit
def kernel(inputs, embeddings):
    batch = inputs.shape[0]
    xt = inputs.T
    nrows = embeddings.shape[0]
    embpad = jnp.concatenate(
        [embeddings, jnp.zeros((nrows, 6), embeddings.dtype)], axis=1)
    grid = _make_grid_kernel(batch)
    out = grid(xt[0], xt[1], xt[2], embpad)
    return out.T


# speedup vs baseline: 1.2143x; 1.2143x over previous
import functools
import math

import jax
import jax.numpy as jnp
import numpy as np
from jax import lax
from jax.experimental import pallas as pl
from jax.experimental.pallas import tpu as pltpu
from jax.experimental.pallas import tpu_sc as plsc

INPUT_DIM = 3
NUM_LEVELS = 16
LEVEL_DIM = 2
BASE_RESOLUTION = 16
LOG2_HASHMAP_SIZE = 19

NC = 2
NS = 16
NW = NC * NS
LANES = 16


def _level_tables():
    offsets = []
    offset = 0
    max_params = 2 ** LOG2_HASHMAP_SIZE
    for i in range(NUM_LEVELS):
        resolution = int(np.ceil(BASE_RESOLUTION * 2.0 ** i))
        params_in_level = min(max_params, resolution ** INPUT_DIM)
        params_in_level = int(np.ceil(params_in_level / 8) * 8)
        offsets.append(offset)
        offset += params_in_level
    offsets.append(offset)

    params = []
    for lvl in range(NUM_LEVELS):
        size = offsets[lvl + 1] - offsets[lvl]
        scale = 2.0 ** lvl * BASE_RESOLUTION - 1.0
        resolution = int(math.ceil(scale)) + 1
        stride = 1
        coeffs = []
        use_stride = []
        for _ in range(INPUT_DIM):
            use_stride.append(stride <= size)
            coeffs.append(stride % (2 ** 32))
            stride = (stride * (resolution + 1)) % (2 ** 32)
        hashed = stride > size
        if hashed:
            c1 = int(np.int32(np.uint32(2654435761)))
            c2 = int(np.int32(np.uint32(805459861)))
            mode_add = False
        else:
            assert all(use_stride)
            c1 = int(np.int32(np.uint32(coeffs[1])))
            c2 = int(np.int32(np.uint32(coeffs[2])))
            mode_add = True
        params.append(dict(scale=float(scale), mask=size - 1,
                           off=offsets[lvl], add=mode_add, c1=c1, c2=c2))
    return params


_LEVELS = _level_tables()


def _make_grid_kernel(batch):
    chunk = batch // NW
    n = 512
    assert chunk % n == 0
    nsub = chunk // n
    groups = n // LANES
    m = 8 * n // 128

    f32 = jnp.float32
    i32 = jnp.int32

    def body(xh, yh, zh, eh, oh,
             xb, yb, zb, fb, idx0, idx1, lo0, lo1, rows0, rows1, acc0, acc1,
             gsem0, gsem1, osem0, osem1):
        cid = lax.axis_index("c")
        sid = lax.axis_index("s")
        wid = sid * NC + cid
        base_w = wid * chunk
        iota = lax.iota(i32, LANES)
        idxb = (idx0, idx1)
        lob = (lo0, lo1)
        rowsb = (rows0, rows1)
        accb = (acc0, acc1)
        gsem = (gsem0, gsem1)
        osem = (osem0, osem1)

        def subchunk(s, carry):
            base = base_w + s * n
            pltpu.sync_copy(xh.at[pl.ds(base, n)], xb)
            pltpu.sync_copy(yh.at[pl.ds(base, n)], yb)
            pltpu.sync_copy(zh.at[pl.ds(base, n)], zb)

            def tbody(g, c):
                o = g * LANES
                for ref in (xb, yb, zb):
                    v = ref[pl.ds(o, LANES)]
                    ref[pl.ds(o, LANES)] = (v + f32(1.0)) * f32(0.5)
                return c
            lax.fori_loop(0, groups, tbody, 0)

            def compute_idx(lvl, p):
                prm = _LEVELS[lvl]
                scale = f32(prm["scale"])
                half = f32(0.5)
                c1 = i32(prm["c1"])
                c2 = i32(prm["c2"])
                mask = i32(prm["mask"])
                off = i32(prm["off"])
                iref = idxb[p]
                lref = lob[p]

                def cbody(g, c):
                    o = g * LANES
                    px = xb[pl.ds(o, LANES)] * scale + half
                    py = yb[pl.ds(o, LANES)] * scale + half
                    pz = zb[pl.ds(o, LANES)] * scale + half
                    pix = px.astype(i32)
                    piy = py.astype(i32)
                    piz = pz.astype(i32)
                    fb[p, 0, pl.ds(o, LANES)] = px - pix.astype(f32)
                    fb[p, 1, pl.ds(o, LANES)] = py - piy.astype(f32)
                    fb[p, 2, pl.ds(o, LANES)] = pz - piz.astype(f32)
                    ax = (pix, pix + i32(1))
                    by = (piy * c1, piy * c1 + c1)
                    cz = (piz * c2, piz * c2 + c2)
                    for c8 in range(8):
                        a = ax[c8 & 1]
                        b = by[(c8 >> 1) & 1]
                        cc = cz[(c8 >> 2) & 1]
                        if prm["add"]:
                            h = a + b + cc
                        else:
                            h = a ^ b ^ cc
                        glob = (h & mask) + off
                        iref[pl.ds(c8 * n + o, LANES)] = glob >> 2
                        lref[pl.ds(c8 * n + o, LANES)] = (glob & 3) * 2
                    return c
                lax.fori_loop(0, groups, cbody, 0)

            def accumulate(lvl, p):
                rref = rowsb[p]
                aref = accb[p]
                lref = lob[p]

                def abody(g, c):
                    o = g * LANES
                    fx = fb[p, 0, pl.ds(o, LANES)]
                    fy = fb[p, 1, pl.ds(o, LANES)]
                    fz = fb[p, 2, pl.ds(o, LANES)]
                    gx = f32(1.0) - fx
                    gy = f32(1.0) - fy
                    gz = f32(1.0) - fz
                    wxy = (gx * gy, fx * gy, gx * fy, fx * fy)
                    jv = o + iota
                    a0 = None
                    a1 = None
                    for c8 in range(8):
                        w = wxy[c8 & 3] * (gz if c8 < 4 else fz)
                        lo = lref[pl.ds(c8 * n + o, LANES)]
                        v0 = plsc.load_gather(rref, [jv + c8 * n, lo])
                        v1 = plsc.load_gather(rref, [jv + c8 * n, lo + i32(1)])
                        t0 = w * v0
                        t1 = w * v1
                        a0 = t0 if a0 is None else a0 + t0
                        a1 = t1 if a1 is None else a1 + t1
                    aref[0, pl.ds(o, LANES)] = a0
                    aref[1, pl.ds(o, LANES)] = a1
                    return c
                lax.fori_loop(0, groups, abody, 0)

            ghandles = [None, None]
            ohandles = [None, None]
            for lvl in range(NUM_LEVELS):
                p = lvl & 1
                compute_idx(lvl, p)
                ghandles[p] = pltpu.async_copy(eh.at[idxb[p]], rowsb[p],
                                               gsem[p])
                if lvl > 0:
                    q = (lvl - 1) & 1
                    ghandles[q].wait()
                    if lvl >= 3:
                        ohandles[q].wait()
                    accumulate(lvl - 1, q)
                    ohandles[q] = pltpu.async_copy(
                        accb[q],
                        oh.at[pl.ds(2 * (lvl - 1), 2), pl.ds(base, n)],
                        osem[q])
            ghandles[1].wait()
            ohandles[1].wait()
            accumulate(NUM_LEVELS - 1, 1)
            ohandles[1] = pltpu.async_copy(
                accb[1],
                oh.at[pl.ds(2 * (NUM_LEVELS - 1), 2), pl.ds(base, n)],
                osem[1])
            ohandles[0].wait()
            ohandles[1].wait()
            return carry

        lax.fori_loop(0, nsub, subchunk, 0)

    mesh = plsc.VectorSubcoreMesh(core_axis_name="c", subcore_axis_name="s")
    return pl.kernel(
        body,
        out_type=jax.ShapeDtypeStruct((NUM_LEVELS * LEVEL_DIM, batch), f32),
        mesh=mesh,
        compiler_params=pltpu.CompilerParams(
            needs_layout_passes=False,
            use_tc_tiling_on_sc=False,
        ),
        scratch_types=[
            pltpu.VMEM((n,), f32),
            pltpu.VMEM((n,), f32),
            pltpu.VMEM((n,), f32),
            pltpu.VMEM((2, 3, n), f32),
            pltpu.VMEM((8 * n,), i32),
            pltpu.VMEM((8 * n,), i32),
            pltpu.VMEM((8 * n,), i32),
            pltpu.VMEM((8 * n,), i32),
            pltpu.VMEM((8 * n, 8), f32),
            pltpu.VMEM((8 * n, 8), f32),
            pltpu.VMEM((LEVEL_DIM, n), f32),
            pltpu.VMEM((LEVEL_DIM, n), f32),
            pltpu.SemaphoreType.DMA,
            pltpu.SemaphoreType.DMA,
            pltpu.SemaphoreType.DMA,
            pltpu.SemaphoreType.DMA,
        ],
    )


@jax.jit
def kernel(inputs, embeddings):
    batch = inputs.shape[0]
    xt = inputs.T
    grid = _make_grid_kernel(batch)
    packed = embeddings.reshape(embeddings.shape[0] // 4, 8)
    out = grid(xt[0], xt[1], xt[2], packed)
    return out.T


# speedup vs baseline: 8.5069x; 7.0055x over previous
import functools
import math

import jax
import jax.numpy as jnp
import numpy as np
from jax import lax
from jax.experimental import pallas as pl
from jax.experimental.pallas import tpu as pltpu
from jax.experimental.pallas import tpu_sc as plsc

INPUT_DIM = 3
NUM_LEVELS = 16
LEVEL_DIM = 2
BASE_RESOLUTION = 16
LOG2_HASHMAP_SIZE = 19

NC = 2
NS = 16
NW = NC * NS
LANES = 16


def _level_tables():
    offsets = []
    offset = 0
    max_params = 2 ** LOG2_HASHMAP_SIZE
    for i in range(NUM_LEVELS):
        resolution = int(np.ceil(BASE_RESOLUTION * 2.0 ** i))
        params_in_level = min(max_params, resolution ** INPUT_DIM)
        params_in_level = int(np.ceil(params_in_level / 8) * 8)
        offsets.append(offset)
        offset += params_in_level
    offsets.append(offset)

    params = []
    for lvl in range(NUM_LEVELS):
        size = offsets[lvl + 1] - offsets[lvl]
        scale = 2.0 ** lvl * BASE_RESOLUTION - 1.0
        resolution = int(math.ceil(scale)) + 1
        stride = 1
        coeffs = []
        use_stride = []
        for _ in range(INPUT_DIM):
            use_stride.append(stride <= size)
            coeffs.append(stride % (2 ** 32))
            stride = (stride * (resolution + 1)) % (2 ** 32)
        hashed = stride > size
        if hashed:
            c1 = int(np.int32(np.uint32(2654435761)))
            c2 = int(np.int32(np.uint32(805459861)))
            mode_add = False
        else:
            assert all(use_stride)
            c1 = int(np.int32(np.uint32(coeffs[1])))
            c2 = int(np.int32(np.uint32(coeffs[2])))
            mode_add = True
        params.append(dict(scale=float(scale), mask=size - 1,
                           off=offsets[lvl], add=mode_add, c1=c1, c2=c2))
    return params


_LEVELS = _level_tables()


def _make_grid_kernel(batch):
    chunk = batch // NW
    n = 256
    assert chunk % n == 0
    nsub = chunk // n
    groups = n // LANES
    m = 8 * n // 128

    f32 = jnp.float32
    i32 = jnp.int32

    def body(xh, yh, zh, eh, oh,
             xb, yb, zb, fb, idx0, idx1, lo0, lo1, rows0, rows1, acc0, acc1,
             gsem0, gsem1, osem0, osem1):
        cid = lax.axis_index("c")
        sid = lax.axis_index("s")
        wid = sid * NC + cid
        base_w = wid * chunk
        iota = lax.iota(i32, LANES)
        idxb = (idx0, idx1)
        lob = (lo0, lo1)
        rowsb = (rows0, rows1)
        accb = (acc0, acc1)
        gsem = (gsem0, gsem1)
        osem = (osem0, osem1)

        def subchunk(s, carry):
            base = base_w + s * n
            pltpu.sync_copy(xh.at[pl.ds(base, n)], xb)
            pltpu.sync_copy(yh.at[pl.ds(base, n)], yb)
            pltpu.sync_copy(zh.at[pl.ds(base, n)], zb)

            def tbody(g, c):
                o = g * LANES
                for ref in (xb, yb, zb):
                    v = ref[pl.ds(o, LANES)]
                    ref[pl.ds(o, LANES)] = (v + f32(1.0)) * f32(0.5)
                return c
            lax.fori_loop(0, groups, tbody, 0)

            def compute_idx(lvl, p):
                prm = _LEVELS[lvl]
                scale = f32(prm["scale"])
                half = f32(0.5)
                c1 = i32(prm["c1"])
                c2 = i32(prm["c2"])
                mask = i32(prm["mask"])
                off = i32(prm["off"])
                iref = idxb[p]
                lref = lob[p]

                def cbody(g, c):
                    o = g * LANES
                    px = xb[pl.ds(o, LANES)] * scale + half
                    py = yb[pl.ds(o, LANES)] * scale + half
                    pz = zb[pl.ds(o, LANES)] * scale + half
                    pix = px.astype(i32)
                    piy = py.astype(i32)
                    piz = pz.astype(i32)
                    fb[p, 0, pl.ds(o, LANES)] = px - pix.astype(f32)
                    fb[p, 1, pl.ds(o, LANES)] = py - piy.astype(f32)
                    fb[p, 2, pl.ds(o, LANES)] = pz - piz.astype(f32)
                    ax = (pix, pix + i32(1))
                    by = (piy * c1, piy * c1 + c1)
                    cz = (piz * c2, piz * c2 + c2)
                    for c8 in range(8):
                        a = ax[c8 & 1]
                        b = by[(c8 >> 1) & 1]
                        cc = cz[(c8 >> 2) & 1]
                        if prm["add"]:
                            h = a + b + cc
                        else:
                            h = a ^ b ^ cc
                        glob = (h & mask) + off
                        p0 = ((glob >> 7) << 5) + ((glob & i32(127)) >> 3)
                        iref[pl.ds(c8 * n + o, LANES)] = p0
                        iref[pl.ds(8 * n + c8 * n + o, LANES)] = p0 + i32(16)
                        lref[pl.ds(c8 * n + o, LANES)] = glob & i32(7)
                    return c
                lax.fori_loop(0, groups, cbody, 0)

            def accumulate(lvl, p):
                rref = rowsb[p]
                aref = accb[p]
                lref = lob[p]

                def abody(g, c):
                    o = g * LANES
                    fx = fb[p, 0, pl.ds(o, LANES)]
                    fy = fb[p, 1, pl.ds(o, LANES)]
                    fz = fb[p, 2, pl.ds(o, LANES)]
                    gx = f32(1.0) - fx
                    gy = f32(1.0) - fy
                    gz = f32(1.0) - fz
                    wxy = (gx * gy, fx * gy, gx * fy, fx * fy)
                    jv = o + iota
                    a0 = None
                    a1 = None
                    for c8 in range(8):
                        w = wxy[c8 & 3] * (gz if c8 < 4 else fz)
                        lo = lref[pl.ds(c8 * n + o, LANES)]
                        v0 = plsc.load_gather(rref, [jv + c8 * n, lo])
                        v1 = plsc.load_gather(rref, [jv + (8 + c8) * n, lo])
                        t0 = w * v0
                        t1 = w * v1
                        a0 = t0 if a0 is None else a0 + t0
                        a1 = t1 if a1 is None else a1 + t1
                    aref[0, pl.ds(o, LANES)] = a0
                    aref[1, pl.ds(o, LANES)] = a1
                    return c
                lax.fori_loop(0, groups, abody, 0)

            ghandles = [None, None]
            ohandles = [None, None]
            for lvl in range(NUM_LEVELS):
                p = lvl & 1
                compute_idx(lvl, p)
                ghandles[p] = pltpu.async_copy(eh.at[idxb[p]], rowsb[p],
                                               gsem[p])
                if lvl > 0:
                    q = (lvl - 1) & 1
                    ghandles[q].wait()
                    if lvl >= 3:
                        ohandles[q].wait()
                    accumulate(lvl - 1, q)
                    ohandles[q] = pltpu.async_copy(
                        accb[q],
                        oh.at[pl.ds(2 * (lvl - 1), 2), pl.ds(base, n)],
                        osem[q])
            ghandles[1].wait()
            ohandles[1].wait()
            accumulate(NUM_LEVELS - 1, 1)
            ohandles[1] = pltpu.async_copy(
                accb[1],
                oh.at[pl.ds(2 * (NUM_LEVELS - 1), 2), pl.ds(base, n)],
                osem[1])
            ohandles[0].wait()
            ohandles[1].wait()
            return carry

        lax.fori_loop(0, nsub, subchunk, 0)

    mesh = plsc.VectorSubcoreMesh(core_axis_name="c", subcore_axis_name="s")
    return pl.kernel(
        body,
        out_type=jax.ShapeDtypeStruct((NUM_LEVELS * LEVEL_DIM, batch), f32),
        mesh=mesh,
        compiler_params=pltpu.CompilerParams(
            needs_layout_passes=False,
            use_tc_tiling_on_sc=False,
        ),
        scratch_types=[
            pltpu.VMEM((n,), f32),
            pltpu.VMEM((n,), f32),
            pltpu.VMEM((n,), f32),
            pltpu.VMEM((2, 3, n), f32),
            pltpu.VMEM((16 * n,), i32),
            pltpu.VMEM((16 * n,), i32),
            pltpu.VMEM((8 * n,), i32),
            pltpu.VMEM((8 * n,), i32),
            pltpu.VMEM((16 * n, 8), f32),
            pltpu.VMEM((16 * n, 8), f32),
            pltpu.VMEM((LEVEL_DIM, n), f32),
            pltpu.VMEM((LEVEL_DIM, n), f32),
            pltpu.SemaphoreType.DMA,
            pltpu.SemaphoreType.DMA,
            pltpu.SemaphoreType.DMA,
            pltpu.SemaphoreType.DMA,
        ],
    )


@jax.jit
def kernel(inputs, embeddings):
    batch = inputs.shape[0]
    xt = inputs.T
    grid = _make_grid_kernel(batch)
    nrows = embeddings.shape[0]
    emb3 = jnp.transpose(embeddings.reshape(nrows // 128, 128, 2), (0, 2, 1))
    packed = emb3.reshape(nrows * 2 // 8, 8)
    out = grid(xt[0], xt[1], xt[2], packed)
    return out.T
